# baseline (speedup 1.0000x reference)
import jax
import jax.numpy as jnp
from jax.experimental import pallas as pl
from jax.experimental.pallas import tpu as pltpu

_MT = 512
_KA = 16


def _split3(v):
    hi = v.astype(jnp.bfloat16)
    r = v - hi.astype(jnp.float32)
    mid = r.astype(jnp.bfloat16)
    lo = (r - mid.astype(jnp.float32)).astype(jnp.bfloat16)
    return hi, mid, lo


def _aug(pts, ref, scale, norms_first):
    bf16 = jnp.bfloat16
    P = pts.shape[0]
    sq = jnp.sum(pts * pts, axis=1, keepdims=True)
    hi, mid, lo = _split3(sq)
    ref[...] = jnp.zeros(ref.shape, bf16)
    ref[:, 0:3] = (scale * pts).astype(bf16)
    if norms_first:
        ref[:, 3:4] = hi
        ref[:, 4:5] = mid
        ref[:, 5:6] = lo
        ref[:, 6:9] = jnp.ones((P, 3), bf16)
    else:
        ref[:, 3:6] = jnp.ones((P, 3), bf16)
        ref[:, 6:7] = hi
        ref[:, 7:8] = mid
        ref[:, 8:9] = lo


def _chamfer_tc_kernel(x_ref, yt_ref, out_ref, xa_ref, ya_ref, rmin_ref):
    x = x_ref[0]
    yt = yt_ref[0]
    bf16 = jnp.bfloat16
    N = x.shape[0]
    M = yt.shape[1]

    _aug(x, xa_ref, -2.0, True)
    ysq = jnp.sum(yt * yt, axis=0, keepdims=True)
    yh, ym, yl = _split3(ysq)
    ya_ref[...] = jnp.zeros(ya_ref.shape, bf16)
    ya_ref[0:3, :] = yt.astype(bf16)
    ya_ref[3:6, :] = jnp.ones((3, M), bf16)
    ya_ref[6:7, :] = yh
    ya_ref[7:8, :] = ym
    ya_ref[8:9, :] = yl

    xa = xa_ref[...]
    col_sums = []
    for j in range(M // _MT):
        dj = jax.lax.dot_general(
            xa, ya_ref[:, pl.ds(j * _MT, _MT)],
            dimension_numbers=(((1,), (0,)), ((), ())),
            preferred_element_type=jnp.float32,
        )
        m = dj[:, 0:128]
        for k in range(1, _MT // 128):
            m = jnp.minimum(m, dj[:, k * 128:(k + 1) * 128])
        if j == 0:
            rmin_ref[...] = m
        else:
            rmin_ref[...] = jnp.minimum(rmin_ref[...], m)
        cm = jnp.min(dj, axis=0, keepdims=True)
        col_sums.append(jnp.sum(cm, axis=1, keepdims=True))

    row_min = jnp.min(rmin_ref[...], axis=1, keepdims=True)

    rt = jnp.sum(row_min, axis=0, keepdims=True)
    ct = sum(col_sums)
    out_ref[0] = jnp.concatenate([rt, ct], axis=1)


def kernel(pred, gt):
    B, N, D = pred.shape
    M = gt.shape[1]
    gt_t = jnp.swapaxes(gt, 1, 2)

    part = pl.pallas_call(
        _chamfer_tc_kernel,
        grid=(B,),
        in_specs=[
            pl.BlockSpec((1, N, D), lambda b: (b, 0, 0)),
            pl.BlockSpec((1, D, M), lambda b: (b, 0, 0)),
        ],
        out_specs=pl.BlockSpec((1, 1, 2), lambda b: (b, 0, 0)),
        out_shape=jax.ShapeDtypeStruct((B, 1, 2), jnp.float32),
        scratch_shapes=[
            pltpu.VMEM((N, _KA), jnp.bfloat16),
            pltpu.VMEM((_KA, M), jnp.bfloat16),
            pltpu.VMEM((N, 128), jnp.float32),
        ],
        compiler_params=pltpu.CompilerParams(
            dimension_semantics=("parallel",),
        ),
    )(pred, gt_t)

    tot = jnp.sum(part, axis=(0, 1))
    return tot[0] / (B * N) + tot[1] / (B * M)


# speedup vs baseline: 1.0364x; 1.0364x over previous
import jax
import jax.numpy as jnp
from jax.experimental import pallas as pl
from jax.experimental.pallas import tpu as pltpu

_MT = 512
_KA = 16


def _split3(v):
    hi = v.astype(jnp.bfloat16)
    r = v - hi.astype(jnp.float32)
    mid = r.astype(jnp.bfloat16)
    lo = (r - mid.astype(jnp.float32)).astype(jnp.bfloat16)
    return hi, mid, lo


def _aug(pts, ref, scale, norms_first):
    bf16 = jnp.bfloat16
    P = pts.shape[0]
    sq = jnp.sum(pts * pts, axis=1, keepdims=True)
    hi, mid, lo = _split3(sq)
    ref[...] = jnp.zeros(ref.shape, bf16)
    ref[:, 0:3] = (scale * pts).astype(bf16)
    if norms_first:
        ref[:, 3:4] = hi
        ref[:, 4:5] = mid
        ref[:, 5:6] = lo
        ref[:, 6:9] = jnp.ones((P, 3), bf16)
    else:
        ref[:, 3:6] = jnp.ones((P, 3), bf16)
        ref[:, 6:7] = hi
        ref[:, 7:8] = mid
        ref[:, 8:9] = lo


def _chamfer_tc_kernel(x_ref, yt_ref, out_ref, xa_ref, ya_ref, acc_ref,
                       rmin_ref):
    b = pl.program_id(0)
    nb = pl.num_programs(0)

    x = x_ref[0]
    yt = yt_ref[0]
    bf16 = jnp.bfloat16
    N = x.shape[0]
    M = yt.shape[1]

    _aug(x, xa_ref, -2.0, True)

    ysq = jnp.sum(yt * yt, axis=0, keepdims=True)
    yh, ym, yl = _split3(ysq)
    ya_ref[...] = jnp.zeros(ya_ref.shape, bf16)
    ya_ref[0:3, :] = yt.astype(bf16)
    ya_ref[3:6, :] = jnp.ones((3, M), bf16)
    ya_ref[6:7, :] = yh
    ya_ref[7:8, :] = ym
    ya_ref[8:9, :] = yl

    @pl.when(b == 0)
    def _init():
        acc_ref[...] = jnp.zeros((1, 2), jnp.float32)

    xa = xa_ref[...]
    col_sums = []
    for j in range(M // _MT):
        dj = jax.lax.dot_general(
            xa, ya_ref[:, pl.ds(j * _MT, _MT)],
            dimension_numbers=(((1,), (0,)), ((), ())),
            preferred_element_type=jnp.float32,
        )
        m = dj[:, 0:128]
        for k in range(1, _MT // 128):
            m = jnp.minimum(m, dj[:, k * 128:(k + 1) * 128])
        if j == 0:
            rmin_ref[...] = m
        else:
            rmin_ref[...] = jnp.minimum(rmin_ref[...], m)
        cm = jnp.min(dj, axis=0, keepdims=True)
        col_sums.append(jnp.sum(cm, axis=1, keepdims=True))

    row_min = jnp.min(rmin_ref[...], axis=1, keepdims=True)

    rt = jnp.sum(row_min, axis=0, keepdims=True)
    ct = sum(col_sums)
    acc_ref[...] += jnp.concatenate([rt, ct], axis=1)

    @pl.when(b == nb - 1)
    def _fini():
        out_ref[...] = (acc_ref[0:1, 0:1] / (nb * N)
                        + acc_ref[0:1, 1:2] / (nb * M))


def kernel(pred, gt):
    B, N, D = pred.shape
    M = gt.shape[1]
    gt_t = jnp.swapaxes(gt, 1, 2)

    out = pl.pallas_call(
        _chamfer_tc_kernel,
        grid=(B,),
        in_specs=[
            pl.BlockSpec((1, N, D), lambda b: (b, 0, 0)),
            pl.BlockSpec((1, D, M), lambda b: (b, 0, 0)),
        ],
        out_specs=pl.BlockSpec((1, 1), lambda b: (0, 0)),
        out_shape=jax.ShapeDtypeStruct((1, 1), jnp.float32),
        scratch_shapes=[
            pltpu.VMEM((N, _KA), jnp.bfloat16),
            pltpu.VMEM((_KA, M), jnp.bfloat16),
            pltpu.VMEM((1, 2), jnp.float32),
            pltpu.VMEM((N, 128), jnp.float32),
        ],
    )(pred, gt_t)

    return out[0, 0]
